# baseline (speedup 1.0000x reference)
import functools

import jax
import jax.numpy as jnp
from jax import lax
from jax.experimental import pallas as pl
from jax.experimental.pallas import tpu as pltpu
from jax.experimental.pallas import tpu_sc as plsc


_CHUNK = 128


def _sc_gather_body(nchunks, table_hbm, idx_hbm, out_hbm, idx_v, rows_v, sem):
    nc = 2
    wid = lax.axis_index("s") * nc + lax.axis_index("c")
    b_per_w = nchunks * _CHUNK
    base = wid * b_per_w
    pltpu.sync_copy(idx_hbm.at[pl.ds(base, b_per_w)], idx_v)

    def chunk(j, carry):
        off = pl.multiple_of(j * _CHUNK, _CHUNK)
        pltpu.async_copy(
            table_hbm.at[idx_v.at[pl.ds(off, _CHUNK)]], rows_v, sem
        ).wait()
        pltpu.sync_copy(rows_v, out_hbm.at[pl.ds(base + off, _CHUNK)])
        return carry

    lax.fori_loop(0, nchunks, chunk, 0)


def _sc_gather(table, ids):
    v, d = table.shape
    (b,) = ids.shape
    nw = 32
    assert b % (nw * _CHUNK) == 0
    nchunks = b // (nw * _CHUNK)
    mesh = plsc.VectorSubcoreMesh(core_axis_name="c", subcore_axis_name="s")
    kern = functools.partial(
        pl.kernel,
        mesh=mesh,
        out_type=jax.ShapeDtypeStruct((b, d), jnp.float32),
        scratch_types=[
            pltpu.VMEM((nchunks * _CHUNK,), jnp.int32),
            pltpu.VMEM((_CHUNK, d), jnp.float32),
            pltpu.SemaphoreType.DMA,
        ],
        compiler_params=pltpu.CompilerParams(use_tc_tiling_on_sc=False),
    )(functools.partial(_sc_gather_body, nchunks))
    return kern(table, ids)


def _proj_body(x_ref, p_ref, o_ref):
    o_ref[...] = jnp.dot(
        x_ref[...], p_ref[...], preferred_element_type=jnp.float32
    )


def _tc_project(x, p):
    r, e = x.shape
    h = p.shape[1]
    bm = 2048
    assert r % bm == 0
    return pl.pallas_call(
        _proj_body,
        grid=(r // bm,),
        in_specs=[
            pl.BlockSpec((bm, e), lambda i: (i, 0)),
            pl.BlockSpec((e, h), lambda i: (0, 0)),
        ],
        out_specs=pl.BlockSpec((bm, h), lambda i: (i, 0)),
        out_shape=jax.ShapeDtypeStruct((r, h), jnp.float32),
    )(x, p)


def kernel(inputs, weight_embedding_table, project_variable):
    batch, seq = inputs.shape
    ids = inputs.reshape(-1).astype(jnp.int32)
    gathered = _sc_gather(weight_embedding_table, ids)
    out = _tc_project(gathered, project_variable)
    return out.reshape(batch, seq, project_variable.shape[1])


# speedup vs baseline: 1.0103x; 1.0103x over previous
import functools

import jax
import jax.numpy as jnp
from jax import lax
from jax.experimental import pallas as pl
from jax.experimental.pallas import tpu as pltpu
from jax.experimental.pallas import tpu_sc as plsc


_CHUNK = 128


def _sc_gather_body(nchunks, table_hbm, idx_hbm, out_hbm, idx_v, rows_v, sem):
    nc = 2
    wid = lax.axis_index("s") * nc + lax.axis_index("c")
    b_per_w = nchunks * _CHUNK
    base = wid * b_per_w
    pltpu.sync_copy(idx_hbm.at[pl.ds(base, b_per_w)], idx_v)

    def chunk(j, carry):
        off = pl.multiple_of(j * _CHUNK, _CHUNK)
        pltpu.async_copy(
            table_hbm.at[idx_v.at[pl.ds(off, _CHUNK)]], rows_v, sem
        ).wait()
        pltpu.sync_copy(rows_v, out_hbm.at[pl.ds(base + off, _CHUNK)])
        return carry

    lax.fori_loop(0, nchunks, chunk, 0)


def _sc_gather(table2, pair_ids):
    v2, d2 = table2.shape
    (b,) = pair_ids.shape
    nw = 32
    assert b % (nw * _CHUNK) == 0
    nchunks = b // (nw * _CHUNK)
    mesh = plsc.VectorSubcoreMesh(core_axis_name="c", subcore_axis_name="s")
    kern = functools.partial(
        pl.kernel,
        mesh=mesh,
        out_type=jax.ShapeDtypeStruct((b, d2), jnp.float32),
        scratch_types=[
            pltpu.VMEM((nchunks * _CHUNK,), jnp.int32),
            pltpu.VMEM((_CHUNK, d2), jnp.float32),
            pltpu.SemaphoreType.DMA,
        ],
    )(functools.partial(_sc_gather_body, nchunks))
    return kern(table2, pair_ids)


def _proj_body(ids_ref, x_ref, p_ref, o_ref):
    e = p_ref.shape[0]
    bm = x_ref.shape[0]
    par = (ids_ref[0, 0, :] & 1).astype(jnp.float32).reshape(bm, 1)
    a = x_ref[:, :e]
    c = x_ref[:, e:]
    xsel = a + (c - a) * par
    o_ref[...] = jnp.dot(xsel, p_ref[...], preferred_element_type=jnp.float32)


def _tc_project(y, ids, p):
    r, d2 = y.shape
    e, h = p.shape
    bm = 2048
    assert r % bm == 0 and d2 == 2 * e
    ids3 = ids.reshape(r // bm, 1, bm)
    return pl.pallas_call(
        _proj_body,
        grid=(r // bm,),
        in_specs=[
            pl.BlockSpec((1, 1, bm), lambda i: (i, 0, 0)),
            pl.BlockSpec((bm, d2), lambda i: (i, 0)),
            pl.BlockSpec((e, h), lambda i: (0, 0)),
        ],
        out_specs=pl.BlockSpec((bm, h), lambda i: (i, 0)),
        out_shape=jax.ShapeDtypeStruct((r, h), jnp.float32),
    )(ids3, y, p)


def kernel(inputs, weight_embedding_table, project_variable):
    batch, seq = inputs.shape
    v, e = weight_embedding_table.shape
    ids = inputs.reshape(-1).astype(jnp.int32)
    table2 = weight_embedding_table.reshape(v // 2, 2 * e)
    y = _sc_gather(table2, ids >> 1)
    out = _tc_project(y, ids, project_variable)
    return out.reshape(batch, seq, project_variable.shape[1])
